# baseline (speedup 1.0000x reference)
import functools

import jax
import jax.numpy as jnp
from jax import lax
from jax.experimental import pallas as pl
from jax.experimental.pallas import tpu as pltpu
from jax.experimental.pallas import tpu_sc as plsc

_CB = 80
_NSUB = 25
_NCORES = 2
_NSUB_CORES = 16
_LANES = 16


@functools.lru_cache(maxsize=None)
def _sc_agg(N, E, R, D):
    nw = _NCORES * _NSUB_CORES
    rows_total = E // _CB
    rows_per_tile = rows_total // nw
    nblocks = rows_per_tile // _NSUB
    nrows_out = N // _NSUB_CORES
    zr = 25
    out_r = 125

    mesh = plsc.VectorSubcoreMesh(core_axis_name="c", subcore_axis_name="s")

    @functools.partial(
        pl.kernel,
        out_type=[
            jax.ShapeDtypeStruct((_NCORES, N, D), jnp.float32),
            jax.ShapeDtypeStruct((_NCORES, N), jnp.float32),
        ],
        mesh=mesh,
        compiler_params=pltpu.CompilerParams(
            use_tc_tiling_on_sc=False, needs_layout_passes=False),
        scratch_types=[
            pltpu.VMEM((_NSUB, _CB), jnp.int32),
            pltpu.VMEM((_NSUB, _CB), jnp.int32),
            pltpu.VMEM((_NSUB, _CB), jnp.int32),
            pltpu.VMEM((_NSUB, _CB), jnp.int32),
            pltpu.VMEM((_CB,), jnp.float32),
            pltpu.VMEM((_CB,), jnp.float32),
            pltpu.VMEM((_CB,), jnp.float32),
            pltpu.VMEM((_CB, D), jnp.float32),
            pltpu.VMEM((_CB, D), jnp.float32),
            pltpu.VMEM((_CB, D), jnp.float32),
            pltpu.VMEM((_CB, R), jnp.float32),
            pltpu.VMEM((_CB, R), jnp.float32),
            pltpu.VMEM((_CB, R), jnp.float32),
            pltpu.VMEM((_CB, R), jnp.float32),
            pltpu.VMEM((_CB, R), jnp.float32),
            pltpu.VMEM((_CB, R), jnp.float32),
            pltpu.VMEM((zr, D), jnp.float32),
            pltpu.VMEM((2000,), jnp.float32),
            pltpu.VMEM_SHARED((N, D), jnp.float32),
            pltpu.VMEM_SHARED((N,), jnp.float32),
            pltpu.SemaphoreType.DMA,
            pltpu.SemaphoreType.DMA,
            pltpu.SemaphoreType.DMA,
            pltpu.SemaphoreType.DMA,
            pltpu.SemaphoreType.DMA,
            pltpu.SemaphoreType.DMA,
        ],
    )
    def agg(src_hbm, dst_hbm, et_hbm, sq_hbm, sk_hbm, xw_hbm, y_out, ss_out,
            srcb, dstb, etb, rowidxb, exb0, exb1, exb2,
            rowsb0, rowsb1, rowsb2, sdb0, sdb1, sdb2, ssb0, ssb1, ssb2,
            zbuf2, zbuf1, y_sh, ssum_sh,
            sem_g0, sem_g1, sem_g2, sem_s0, sem_s1, sem_s2):
        exb = [exb0, exb1, exb2]
        rowsb = [rowsb0, rowsb1, rowsb2]
        sdb = [sdb0, sdb1, sdb2]
        ssb = [ssb0, ssb1, ssb2]
        sem_g = [sem_g0, sem_g1, sem_g2]
        sem_s = [sem_s0, sem_s1, sem_s2]
        c = lax.axis_index("c")
        s = lax.axis_index("s")
        wid = c * _NSUB_CORES + s
        lane = lax.iota(jnp.int32, _LANES)

        zv = jnp.zeros((_LANES,), jnp.float32)

        def _z2(i, carry):
            zbuf2[lax.div(i, 8), pl.ds(lax.rem(i, 8) * _LANES, _LANES)] = zv
            return carry
        lax.fori_loop(0, zr * (D // _LANES), _z2, 0)

        def _z1(i, carry):
            zbuf1[pl.ds(i * _LANES, _LANES)] = zv
            return carry
        lax.fori_loop(0, 2000 // _LANES, _z1, 0)

        for t in range(nrows_out // zr):
            pltpu.sync_copy(zbuf2, y_sh.at[pl.ds(s * nrows_out + t * zr, zr)])

        @pl.when(s == 0)
        def _():
            for t in range(N // 2000):
                pltpu.sync_copy(zbuf1, ssum_sh.at[pl.ds(t * 2000, 2000)])

        plsc.subcore_barrier()

        def issue_gathers(j, b):
            for j2 in range(_CB // _LANES):
                sl = pl.ds(j2 * _LANES, _LANES)
                rowidxb[j, sl] = etb[j, sl] * N + srcb[j, sl]
            pltpu.async_copy(xw_hbm.at[rowidxb.at[j]], rowsb[b], sem_g[b])
            pltpu.async_copy(sq_hbm.at[dstb.at[j]], sdb[b], sem_g[b])
            pltpu.async_copy(sk_hbm.at[srcb.at[j]], ssb[b], sem_g[b])

        def wait_gathers(j, b):
            pltpu.make_async_copy(xw_hbm.at[rowidxb.at[j]], rowsb[b], sem_g[b]).wait()
            pltpu.make_async_copy(sq_hbm.at[dstb.at[j]], sdb[b], sem_g[b]).wait()
            pltpu.make_async_copy(sk_hbm.at[srcb.at[j]], ssb[b], sem_g[b]).wait()

        def issue_scatters(j, b):
            pltpu.async_copy(exb[b], ssum_sh.at[dstb.at[j]], sem_s[b], add=True)
            pltpu.async_copy(rowsb[b], y_sh.at[dstb.at[j]], sem_s[b], add=True)

        def drain_scatters(j, b):
            pltpu.make_async_copy(exb[b], ssum_sh.at[dstb.at[j]], sem_s[b]).wait()
            pltpu.make_async_copy(rowsb[b], y_sh.at[dstb.at[j]], sem_s[b]).wait()

        def compute(j, b):
            for j2 in range(_CB // _LANES):
                sl = pl.ds(j2 * _LANES, _LANES)
                iv = lane + j2 * _LANES
                etv = etb[j, sl]
                sqv = plsc.load_gather(sdb[b], [iv, etv])
                skv = plsc.load_gather(ssb[b], [iv, etv])
                al = sqv + skv
                al = jnp.where(al >= 0.0, al, al * 0.2)
                exb[b][sl] = jnp.exp(al)

            @plsc.parallel_loop(0, _CB, 1, unroll=8)
            def _(i):
                a = plsc.load_gather(exb[b], [jnp.broadcast_to(i, (_LANES,))])
                for h in range(D // _LANES):
                    hsl = pl.ds(h * _LANES, _LANES)
                    rowsb[b][i, hsl] = rowsb[b][i, hsl] * a

        def step(j, b, drain_j, gather_j):
            wait_gathers(j, b)
            compute(j, b)
            issue_scatters(j, b)
            if drain_j is not None:
                drain_scatters(drain_j, (3 + (b - 1)) % 3)
            if gather_j is not None:
                issue_gathers(gather_j, (3 + (b - 1)) % 3)

        def blk_body(bnum, carry):
            rbase = wid * rows_per_tile + bnum * _NSUB
            pltpu.sync_copy(src_hbm.at[pl.ds(rbase, _NSUB)], srcb)
            pltpu.sync_copy(dst_hbm.at[pl.ds(rbase, _NSUB)], dstb)
            pltpu.sync_copy(et_hbm.at[pl.ds(rbase, _NSUB)], etb)

            issue_gathers(0, 0)
            issue_gathers(1, 1)
            step(0, 0, None, 2)
            step(1, 1, 0, 3)
            step(2, 2, 1, 4)

            def ring_body(k, carry2):
                j0 = 3 * k
                step(j0, 0, j0 - 1, j0 + 2)
                step(j0 + 1, 1, j0, j0 + 3)
                step(j0 + 2, 2, j0 + 1, j0 + 4)
                return carry2
            lax.fori_loop(1, 1 + (_NSUB - 5) // 3, ring_body, 0)

            step(21, 0, 20, 23)
            step(22, 1, 21, 24)
            step(23, 2, 22, None)
            step(24, 0, 23, None)
            drain_scatters(24, 0)
            return carry
        lax.fori_loop(0, nblocks, blk_body, 0)

        plsc.subcore_barrier()

        for t in range(nrows_out // out_r):
            r0 = s * nrows_out + t * out_r
            pltpu.sync_copy(y_sh.at[pl.ds(r0, out_r)], y_out.at[c, pl.ds(r0, out_r)])

        @pl.when(s == 0)
        def _():
            pltpu.sync_copy(ssum_sh, ss_out.at[c])

    return agg


def _prep(x, W, q, k):
    N, D = x.shape
    R = W.shape[0]
    bn = 1000

    def kern(x_ref, w_ref, q_ref, k_ref, xw_ref, sq_ref, sk_ref):
        xb = x_ref[...]
        sq_cols = []
        sk_cols = []
        for r in range(R):
            xwr = jnp.dot(xb, w_ref[r], preferred_element_type=jnp.float32)
            xw_ref[r] = xwr
            sq_cols.append(jnp.dot(xwr, q_ref[...], preferred_element_type=jnp.float32))
            sk_cols.append(jnp.dot(xwr, k_ref[...], preferred_element_type=jnp.float32))
        sq_ref[...] = jnp.concatenate(sq_cols, axis=1)
        sk_ref[...] = jnp.concatenate(sk_cols, axis=1)

    xw, sq, sk = pl.pallas_call(
        kern,
        grid=(N // bn,),
        in_specs=[
            pl.BlockSpec((bn, D), lambda i: (i, 0)),
            pl.BlockSpec((R, D, D), lambda i: (0, 0, 0)),
            pl.BlockSpec((D, 1), lambda i: (0, 0)),
            pl.BlockSpec((D, 1), lambda i: (0, 0)),
        ],
        out_specs=[
            pl.BlockSpec((R, bn, D), lambda i: (0, i, 0)),
            pl.BlockSpec((bn, R), lambda i: (i, 0)),
            pl.BlockSpec((bn, R), lambda i: (i, 0)),
        ],
        out_shape=[
            jax.ShapeDtypeStruct((R, N, D), jnp.float32),
            jax.ShapeDtypeStruct((N, R), jnp.float32),
            jax.ShapeDtypeStruct((N, R), jnp.float32),
        ],
    )(x, W, q, k)
    return xw, sq, sk


def _finish(y_part, ss_part, b):
    _, N, D = y_part.shape
    bn = 1000

    def kern(y_ref, ss_ref, b_ref, o_ref):
        ssum = ss_ref[0, 0] + ss_ref[0, 1]
        denom = ssum + 1e-16
        y = y_ref[0] + y_ref[1]
        o = y / denom[:, None] + b_ref[...][None, :]
        o_ref[...] = jnp.maximum(o, 0.0)

    return pl.pallas_call(
        kern,
        grid=(N // bn,),
        in_specs=[
            pl.BlockSpec((2, bn, D), lambda i: (0, i, 0)),
            pl.BlockSpec((1, 2, bn), lambda i: (i, 0, 0)),
            pl.BlockSpec((D,), lambda i: (0,)),
        ],
        out_specs=pl.BlockSpec((bn, D), lambda i: (i, 0)),
        out_shape=jax.ShapeDtypeStruct((N, D), jnp.float32),
    )(y_part, ss_part.reshape(2, N // bn, bn).transpose(1, 0, 2), b)


def _finish_prep(y_part, ss_part, bb, x, W, q, k):
    _, N, D = y_part.shape
    R = W.shape[0]
    bn = 1000

    def kern(y_ref, ss_ref, b_ref, x_ref, w_ref, q_ref, k_ref,
             o_ref, xw_ref, sq_ref, sk_ref):
        ssum = ss_ref[0, 0] + ss_ref[0, 1]
        denom = ssum + 1e-16
        y = y_ref[0] + y_ref[1]
        o = y / denom[:, None] + b_ref[...][None, :]
        o_ref[...] = jnp.maximum(o, 0.0)
        xb = x_ref[...]
        sq_cols = []
        sk_cols = []
        for r in range(R):
            xwr = jnp.dot(xb, w_ref[r], preferred_element_type=jnp.float32)
            xw_ref[r] = xwr
            sq_cols.append(jnp.dot(xwr, q_ref[...], preferred_element_type=jnp.float32))
            sk_cols.append(jnp.dot(xwr, k_ref[...], preferred_element_type=jnp.float32))
        sq_ref[...] = jnp.concatenate(sq_cols, axis=1)
        sk_ref[...] = jnp.concatenate(sk_cols, axis=1)

    return pl.pallas_call(
        kern,
        grid=(N // bn,),
        in_specs=[
            pl.BlockSpec((2, bn, D), lambda i: (0, i, 0)),
            pl.BlockSpec((1, 2, bn), lambda i: (i, 0, 0)),
            pl.BlockSpec((D,), lambda i: (0,)),
            pl.BlockSpec((bn, D), lambda i: (i, 0)),
            pl.BlockSpec((R, D, D), lambda i: (0, 0, 0)),
            pl.BlockSpec((D, 1), lambda i: (0, 0)),
            pl.BlockSpec((D, 1), lambda i: (0, 0)),
        ],
        out_specs=[
            pl.BlockSpec((bn, D), lambda i: (i, 0)),
            pl.BlockSpec((R, bn, D), lambda i: (0, i, 0)),
            pl.BlockSpec((bn, R), lambda i: (i, 0)),
            pl.BlockSpec((bn, R), lambda i: (i, 0)),
        ],
        out_shape=[
            jax.ShapeDtypeStruct((N, D), jnp.float32),
            jax.ShapeDtypeStruct((R, N, D), jnp.float32),
            jax.ShapeDtypeStruct((N, R), jnp.float32),
            jax.ShapeDtypeStruct((N, R), jnp.float32),
        ],
    )(y_part, ss_part.reshape(2, N // bn, bn).transpose(1, 0, 2), bb,
      x, W, q, k)


def _edges(edge_index, edge_type):
    E = edge_index.shape[1]
    rows = E // _CB
    return (edge_index[0].reshape(rows, _CB), edge_index[1].reshape(rows, _CB),
            edge_type.reshape(rows, _CB))


def _conv(x, edge_index, edge_type, W, q, k, b):
    N, D = x.shape
    E = edge_index.shape[1]
    R = W.shape[0]
    xw, sq, sk = _prep(x, W, q, k)
    rows = E // _CB
    src2 = edge_index[0].reshape(rows, _CB)
    dst2 = edge_index[1].reshape(rows, _CB)
    et2 = edge_type.reshape(rows, _CB)
    y_part, ss_part = _sc_agg(N, E, R, D)(
        src2, dst2, et2, sq, sk, xw.reshape(R * N, D))
    return _finish(y_part, ss_part, b)


def kernel(x_one, edge_index_one, edge_type_one, x_two, edge_index_two,
           edge_type_two, W1, q1, k1, b1, W2, q2, k2, b2):
    N, D = x_one.shape
    E = edge_index_one.shape[1]
    R = W1.shape[0]
    agg = _sc_agg(N, E, R, D)
    xw1, sq1, sk1 = _prep(x_one, W1, q1, k1)
    src1, dst1, et1 = _edges(edge_index_one, edge_type_one)
    y1, ss1 = agg(src1, dst1, et1, sq1, sk1, xw1.reshape(R * N, D))
    h1, xw2, sq2, sk2 = _finish_prep(y1, ss1, b1, x_two, W2, q2, k2)
    src2, dst2, et2 = _edges(edge_index_two, edge_type_two)
    y2, ss2 = agg(src2, dst2, et2, sq2, sk2, xw2.reshape(R * N, D))
    h2 = _finish(y2, ss2, b2)
    return (h1, h2)


# speedup vs baseline: 1.0160x; 1.0160x over previous
import functools

import jax
import jax.numpy as jnp
from jax import lax
from jax.experimental import pallas as pl
from jax.experimental.pallas import tpu as pltpu
from jax.experimental.pallas import tpu_sc as plsc

_CB = 80
_NSUB = 25
_NCORES = 2
_NSUB_CORES = 16
_LANES = 16


@functools.lru_cache(maxsize=None)
def _sc_agg(N, E, R, D):
    nw = _NCORES * _NSUB_CORES
    rows_total = E // _CB
    rows_per_tile = rows_total // nw
    nblocks = rows_per_tile // _NSUB
    nrows_out = N // _NSUB_CORES
    zr = 25
    out_r = 125

    mesh = plsc.VectorSubcoreMesh(core_axis_name="c", subcore_axis_name="s")

    @functools.partial(
        pl.kernel,
        out_type=[
            jax.ShapeDtypeStruct((_NCORES, N, D), jnp.float32),
            jax.ShapeDtypeStruct((_NCORES, N), jnp.float32),
        ],
        mesh=mesh,
        compiler_params=pltpu.CompilerParams(
            use_tc_tiling_on_sc=False, needs_layout_passes=False),
        scratch_types=[
            pltpu.VMEM((_NSUB, _CB), jnp.int32),
            pltpu.VMEM((_NSUB, _CB), jnp.int32),
            pltpu.VMEM((_NSUB, _CB), jnp.int32),
            pltpu.VMEM((_NSUB, _CB), jnp.int32),
            pltpu.VMEM((_CB,), jnp.float32),
            pltpu.VMEM((_CB,), jnp.float32),
            pltpu.VMEM((_CB,), jnp.float32),
            pltpu.VMEM((_CB, D), jnp.float32),
            pltpu.VMEM((_CB, D), jnp.float32),
            pltpu.VMEM((_CB, D), jnp.float32),
            pltpu.VMEM((_CB, R), jnp.float32),
            pltpu.VMEM((_CB, R), jnp.float32),
            pltpu.VMEM((_CB, R), jnp.float32),
            pltpu.VMEM((_CB, R), jnp.float32),
            pltpu.VMEM((_CB, R), jnp.float32),
            pltpu.VMEM((_CB, R), jnp.float32),
            pltpu.VMEM((zr, D), jnp.float32),
            pltpu.VMEM((2000,), jnp.float32),
            pltpu.VMEM_SHARED((N, D), jnp.float32),
            pltpu.VMEM_SHARED((N,), jnp.float32),
            pltpu.SemaphoreType.DMA,
            pltpu.SemaphoreType.DMA,
            pltpu.SemaphoreType.DMA,
            pltpu.SemaphoreType.DMA,
            pltpu.SemaphoreType.DMA,
            pltpu.SemaphoreType.DMA,
        ],
    )
    def agg(src_hbm, dst_hbm, et_hbm, sq_hbm, sk_hbm, xw_hbm, y_out, ss_out,
            srcb, dstb, etb, rowidxb, exb0, exb1, exb2,
            rowsb0, rowsb1, rowsb2, sdb0, sdb1, sdb2, ssb0, ssb1, ssb2,
            zbuf2, zbuf1, y_sh, ssum_sh,
            sem_g0, sem_g1, sem_g2, sem_s0, sem_s1, sem_s2):
        exb = [exb0, exb1, exb2]
        rowsb = [rowsb0, rowsb1, rowsb2]
        sdb = [sdb0, sdb1, sdb2]
        ssb = [ssb0, ssb1, ssb2]
        sem_g = [sem_g0, sem_g1, sem_g2]
        sem_s = [sem_s0, sem_s1, sem_s2]
        c = lax.axis_index("c")
        s = lax.axis_index("s")
        wid = c * _NSUB_CORES + s
        lane = lax.iota(jnp.int32, _LANES)

        zv = jnp.zeros((_LANES,), jnp.float32)

        def _z2(i, carry):
            zbuf2[lax.div(i, 8), pl.ds(lax.rem(i, 8) * _LANES, _LANES)] = zv
            return carry
        lax.fori_loop(0, zr * (D // _LANES), _z2, 0)

        def _z1(i, carry):
            zbuf1[pl.ds(i * _LANES, _LANES)] = zv
            return carry
        lax.fori_loop(0, 2000 // _LANES, _z1, 0)

        for t in range(nrows_out // zr):
            pltpu.sync_copy(zbuf2, y_sh.at[pl.ds(s * nrows_out + t * zr, zr)])

        @pl.when(s == 0)
        def _():
            for t in range(N // 2000):
                pltpu.sync_copy(zbuf1, ssum_sh.at[pl.ds(t * 2000, 2000)])

        plsc.subcore_barrier()

        def issue_gathers(j, b):
            for j2 in range(_CB // _LANES):
                sl = pl.ds(j2 * _LANES, _LANES)
                rowidxb[j, sl] = etb[j, sl] * N + srcb[j, sl]
            pltpu.async_copy(xw_hbm.at[rowidxb.at[j]], rowsb[b], sem_g[b])
            pltpu.async_copy(sq_hbm.at[dstb.at[j]], sdb[b], sem_g[b])
            pltpu.async_copy(sk_hbm.at[srcb.at[j]], ssb[b], sem_g[b])

        def wait_gathers(j, b):
            pltpu.make_async_copy(xw_hbm.at[rowidxb.at[j]], rowsb[b], sem_g[b]).wait()
            pltpu.make_async_copy(sq_hbm.at[dstb.at[j]], sdb[b], sem_g[b]).wait()
            pltpu.make_async_copy(sk_hbm.at[srcb.at[j]], ssb[b], sem_g[b]).wait()

        def issue_scatters(j, b):
            pltpu.async_copy(exb[b], ssum_sh.at[dstb.at[j]], sem_s[b], add=True)
            pltpu.async_copy(rowsb[b], y_sh.at[dstb.at[j]], sem_s[b], add=True)

        def drain_scatters(j, b):
            pltpu.make_async_copy(exb[b], ssum_sh.at[dstb.at[j]], sem_s[b]).wait()
            pltpu.make_async_copy(rowsb[b], y_sh.at[dstb.at[j]], sem_s[b]).wait()

        def compute(j, b):
            for j2 in range(_CB // _LANES):
                sl = pl.ds(j2 * _LANES, _LANES)
                iv = lane + j2 * _LANES
                etv = etb[j, sl]
                sqv = plsc.load_gather(sdb[b], [iv, etv])
                skv = plsc.load_gather(ssb[b], [iv, etv])
                al = sqv + skv
                al = jnp.where(al >= 0.0, al, al * 0.2)
                exb[b][sl] = jnp.exp(al)

            @plsc.parallel_loop(0, _CB, 1, unroll=4)
            def _(i):
                a = plsc.load_gather(exb[b], [jnp.broadcast_to(i, (_LANES,))])
                for h in range(D // _LANES):
                    hsl = pl.ds(h * _LANES, _LANES)
                    rowsb[b][i, hsl] = rowsb[b][i, hsl] * a

        def step(j, b, drain_j, gather_j):
            wait_gathers(j, b)
            compute(j, b)
            issue_scatters(j, b)
            if drain_j is not None:
                drain_scatters(drain_j, (3 + (b - 1)) % 3)
            if gather_j is not None:
                issue_gathers(gather_j, (3 + (b - 1)) % 3)

        def blk_body(bnum, carry):
            rbase = wid * rows_per_tile + bnum * _NSUB
            pltpu.sync_copy(src_hbm.at[pl.ds(rbase, _NSUB)], srcb)
            pltpu.sync_copy(dst_hbm.at[pl.ds(rbase, _NSUB)], dstb)
            pltpu.sync_copy(et_hbm.at[pl.ds(rbase, _NSUB)], etb)

            issue_gathers(0, 0)
            issue_gathers(1, 1)
            step(0, 0, None, 2)
            step(1, 1, 0, 3)
            step(2, 2, 1, 4)

            def ring_body(k, carry2):
                j0 = 3 * k
                step(j0, 0, j0 - 1, j0 + 2)
                step(j0 + 1, 1, j0, j0 + 3)
                step(j0 + 2, 2, j0 + 1, j0 + 4)
                return carry2
            lax.fori_loop(1, 1 + (_NSUB - 5) // 3, ring_body, 0)

            step(21, 0, 20, 23)
            step(22, 1, 21, 24)
            step(23, 2, 22, None)
            step(24, 0, 23, None)
            drain_scatters(24, 0)
            return carry
        lax.fori_loop(0, nblocks, blk_body, 0)

        plsc.subcore_barrier()

        for t in range(nrows_out // out_r):
            r0 = s * nrows_out + t * out_r
            pltpu.sync_copy(y_sh.at[pl.ds(r0, out_r)], y_out.at[c, pl.ds(r0, out_r)])

        @pl.when(s == 0)
        def _():
            pltpu.sync_copy(ssum_sh, ss_out.at[c])

    return agg


def _prep(x, W, q, k):
    N, D = x.shape
    R = W.shape[0]
    bn = 1000

    def kern(x_ref, w_ref, q_ref, k_ref, xw_ref, sq_ref, sk_ref):
        xb = x_ref[...]
        sq_cols = []
        sk_cols = []
        for r in range(R):
            xwr = jnp.dot(xb, w_ref[r], preferred_element_type=jnp.float32)
            xw_ref[r] = xwr
            sq_cols.append(jnp.dot(xwr, q_ref[...], preferred_element_type=jnp.float32))
            sk_cols.append(jnp.dot(xwr, k_ref[...], preferred_element_type=jnp.float32))
        sq_ref[...] = jnp.concatenate(sq_cols, axis=1)
        sk_ref[...] = jnp.concatenate(sk_cols, axis=1)

    xw, sq, sk = pl.pallas_call(
        kern,
        grid=(N // bn,),
        in_specs=[
            pl.BlockSpec((bn, D), lambda i: (i, 0)),
            pl.BlockSpec((R, D, D), lambda i: (0, 0, 0)),
            pl.BlockSpec((D, 1), lambda i: (0, 0)),
            pl.BlockSpec((D, 1), lambda i: (0, 0)),
        ],
        out_specs=[
            pl.BlockSpec((R, bn, D), lambda i: (0, i, 0)),
            pl.BlockSpec((bn, R), lambda i: (i, 0)),
            pl.BlockSpec((bn, R), lambda i: (i, 0)),
        ],
        out_shape=[
            jax.ShapeDtypeStruct((R, N, D), jnp.float32),
            jax.ShapeDtypeStruct((N, R), jnp.float32),
            jax.ShapeDtypeStruct((N, R), jnp.float32),
        ],
    )(x, W, q, k)
    return xw, sq, sk


def _finish(y_part, ss_part, b):
    _, N, D = y_part.shape
    bn = 1000

    def kern(y_ref, ss_ref, b_ref, o_ref):
        ssum = ss_ref[0, 0] + ss_ref[0, 1]
        denom = ssum + 1e-16
        y = y_ref[0] + y_ref[1]
        o = y / denom[:, None] + b_ref[...][None, :]
        o_ref[...] = jnp.maximum(o, 0.0)

    return pl.pallas_call(
        kern,
        grid=(N // bn,),
        in_specs=[
            pl.BlockSpec((2, bn, D), lambda i: (0, i, 0)),
            pl.BlockSpec((1, 2, bn), lambda i: (i, 0, 0)),
            pl.BlockSpec((D,), lambda i: (0,)),
        ],
        out_specs=pl.BlockSpec((bn, D), lambda i: (i, 0)),
        out_shape=jax.ShapeDtypeStruct((N, D), jnp.float32),
    )(y_part, ss_part.reshape(2, N // bn, bn).transpose(1, 0, 2), b)


def _finish_prep(y_part, ss_part, bb, x, W, q, k):
    _, N, D = y_part.shape
    R = W.shape[0]
    bn = 1000

    def kern(y_ref, ss_ref, b_ref, x_ref, w_ref, q_ref, k_ref,
             o_ref, xw_ref, sq_ref, sk_ref):
        ssum = ss_ref[0, 0] + ss_ref[0, 1]
        denom = ssum + 1e-16
        y = y_ref[0] + y_ref[1]
        o = y / denom[:, None] + b_ref[...][None, :]
        o_ref[...] = jnp.maximum(o, 0.0)
        xb = x_ref[...]
        sq_cols = []
        sk_cols = []
        for r in range(R):
            xwr = jnp.dot(xb, w_ref[r], preferred_element_type=jnp.float32)
            xw_ref[r] = xwr
            sq_cols.append(jnp.dot(xwr, q_ref[...], preferred_element_type=jnp.float32))
            sk_cols.append(jnp.dot(xwr, k_ref[...], preferred_element_type=jnp.float32))
        sq_ref[...] = jnp.concatenate(sq_cols, axis=1)
        sk_ref[...] = jnp.concatenate(sk_cols, axis=1)

    return pl.pallas_call(
        kern,
        grid=(N // bn,),
        in_specs=[
            pl.BlockSpec((2, bn, D), lambda i: (0, i, 0)),
            pl.BlockSpec((1, 2, bn), lambda i: (i, 0, 0)),
            pl.BlockSpec((D,), lambda i: (0,)),
            pl.BlockSpec((bn, D), lambda i: (i, 0)),
            pl.BlockSpec((R, D, D), lambda i: (0, 0, 0)),
            pl.BlockSpec((D, 1), lambda i: (0, 0)),
            pl.BlockSpec((D, 1), lambda i: (0, 0)),
        ],
        out_specs=[
            pl.BlockSpec((bn, D), lambda i: (i, 0)),
            pl.BlockSpec((R, bn, D), lambda i: (0, i, 0)),
            pl.BlockSpec((bn, R), lambda i: (i, 0)),
            pl.BlockSpec((bn, R), lambda i: (i, 0)),
        ],
        out_shape=[
            jax.ShapeDtypeStruct((N, D), jnp.float32),
            jax.ShapeDtypeStruct((R, N, D), jnp.float32),
            jax.ShapeDtypeStruct((N, R), jnp.float32),
            jax.ShapeDtypeStruct((N, R), jnp.float32),
        ],
    )(y_part, ss_part.reshape(2, N // bn, bn).transpose(1, 0, 2), bb,
      x, W, q, k)


def _edges(edge_index, edge_type):
    E = edge_index.shape[1]
    rows = E // _CB
    return (edge_index[0].reshape(rows, _CB), edge_index[1].reshape(rows, _CB),
            edge_type.reshape(rows, _CB))


def _conv(x, edge_index, edge_type, W, q, k, b):
    N, D = x.shape
    E = edge_index.shape[1]
    R = W.shape[0]
    xw, sq, sk = _prep(x, W, q, k)
    rows = E // _CB
    src2 = edge_index[0].reshape(rows, _CB)
    dst2 = edge_index[1].reshape(rows, _CB)
    et2 = edge_type.reshape(rows, _CB)
    y_part, ss_part = _sc_agg(N, E, R, D)(
        src2, dst2, et2, sq, sk, xw.reshape(R * N, D))
    return _finish(y_part, ss_part, b)


def kernel(x_one, edge_index_one, edge_type_one, x_two, edge_index_two,
           edge_type_two, W1, q1, k1, b1, W2, q2, k2, b2):
    N, D = x_one.shape
    E = edge_index_one.shape[1]
    R = W1.shape[0]
    agg = _sc_agg(N, E, R, D)
    xw1, sq1, sk1 = _prep(x_one, W1, q1, k1)
    src1, dst1, et1 = _edges(edge_index_one, edge_type_one)
    y1, ss1 = agg(src1, dst1, et1, sq1, sk1, xw1.reshape(R * N, D))
    h1, xw2, sq2, sk2 = _finish_prep(y1, ss1, b1, x_two, W2, q2, k2)
    src2, dst2, et2 = _edges(edge_index_two, edge_type_two)
    y2, ss2 = agg(src2, dst2, et2, sq2, sk2, xw2.reshape(R * N, D))
    h2 = _finish(y2, ss2, b2)
    return (h1, h2)


# speedup vs baseline: 1.1010x; 1.0837x over previous
import functools

import jax
import jax.numpy as jnp
from jax import lax
from jax.experimental import pallas as pl
from jax.experimental.pallas import tpu as pltpu
from jax.experimental.pallas import tpu_sc as plsc

_CB = 80
_NSUB = 25
_NCORES = 2
_NSUB_CORES = 16
_LANES = 16


@functools.lru_cache(maxsize=None)
def _sc_agg(N, E, R, D):
    nw = _NCORES * _NSUB_CORES
    rows_total = E // _CB
    rows_per_tile = rows_total // nw
    nblocks = rows_per_tile // _NSUB
    nrows_out = N // _NSUB_CORES
    zr = 25
    out_r = 125

    mesh = plsc.VectorSubcoreMesh(core_axis_name="c", subcore_axis_name="s")

    @functools.partial(
        pl.kernel,
        out_type=[
            jax.ShapeDtypeStruct((_NCORES, N, D), jnp.float32),
            jax.ShapeDtypeStruct((_NCORES, N), jnp.float32),
        ],
        mesh=mesh,
        compiler_params=pltpu.CompilerParams(
            use_tc_tiling_on_sc=False, needs_layout_passes=False),
        scratch_types=[
            pltpu.VMEM((_NSUB, _CB), jnp.int32),
            pltpu.VMEM((_NSUB, _CB), jnp.int32),
            pltpu.VMEM((_NSUB, _CB), jnp.int32),
            pltpu.VMEM((_NSUB, _CB), jnp.int32),
            pltpu.VMEM((_CB,), jnp.float32),
            pltpu.VMEM((_CB,), jnp.float32),
            pltpu.VMEM((_CB,), jnp.float32),
            pltpu.VMEM((_CB, D), jnp.float32),
            pltpu.VMEM((_CB, D), jnp.float32),
            pltpu.VMEM((_CB, D), jnp.float32),
            pltpu.VMEM((_CB, R), jnp.float32),
            pltpu.VMEM((_CB, R), jnp.float32),
            pltpu.VMEM((_CB, R), jnp.float32),
            pltpu.VMEM((_CB, R), jnp.float32),
            pltpu.VMEM((_CB, R), jnp.float32),
            pltpu.VMEM((_CB, R), jnp.float32),
            pltpu.VMEM((zr, D), jnp.float32),
            pltpu.VMEM((2000,), jnp.float32),
            pltpu.VMEM_SHARED((N, D), jnp.float32),
            pltpu.VMEM_SHARED((N,), jnp.float32),
            pltpu.SemaphoreType.DMA,
            pltpu.SemaphoreType.DMA,
            pltpu.SemaphoreType.DMA,
            pltpu.SemaphoreType.DMA,
            pltpu.SemaphoreType.DMA,
            pltpu.SemaphoreType.DMA,
        ],
    )
    def agg(src_hbm, dst_hbm, et_hbm, sq_hbm, sk_hbm, xw_hbm, y_out, ss_out,
            srcb, dstb, etb, rowidxb, exb0, exb1, exb2,
            rowsb0, rowsb1, rowsb2, sdb0, sdb1, sdb2, ssb0, ssb1, ssb2,
            zbuf2, zbuf1, y_sh, ssum_sh,
            sem_g0, sem_g1, sem_g2, sem_s0, sem_s1, sem_s2):
        exb = [exb0, exb1, exb2]
        rowsb = [rowsb0, rowsb1, rowsb2]
        sdb = [sdb0, sdb1, sdb2]
        ssb = [ssb0, ssb1, ssb2]
        sem_g = [sem_g0, sem_g1, sem_g2]
        sem_s = [sem_s0, sem_s1, sem_s2]
        c = lax.axis_index("c")
        s = lax.axis_index("s")
        wid = c * _NSUB_CORES + s
        lane = lax.iota(jnp.int32, _LANES)

        zv = jnp.zeros((_LANES,), jnp.float32)

        def _z2(i, carry):
            zbuf2[lax.div(i, 8), pl.ds(lax.rem(i, 8) * _LANES, _LANES)] = zv
            return carry
        lax.fori_loop(0, zr * (D // _LANES), _z2, 0)

        def _z1(i, carry):
            zbuf1[pl.ds(i * _LANES, _LANES)] = zv
            return carry
        lax.fori_loop(0, 2000 // _LANES, _z1, 0)

        for t in range(nrows_out // zr):
            pltpu.sync_copy(zbuf2, y_sh.at[pl.ds(s * nrows_out + t * zr, zr)])

        @pl.when(s == 0)
        def _():
            for t in range(N // 2000):
                pltpu.sync_copy(zbuf1, ssum_sh.at[pl.ds(t * 2000, 2000)])

        plsc.subcore_barrier()

        def issue_gathers(j, b):
            for j2 in range(_CB // _LANES):
                sl = pl.ds(j2 * _LANES, _LANES)
                rowidxb[j, sl] = etb[j, sl] * N + srcb[j, sl]
            pltpu.async_copy(xw_hbm.at[rowidxb.at[j]], rowsb[b], sem_g[b])
            pltpu.async_copy(sq_hbm.at[dstb.at[j]], sdb[b], sem_g[b])
            pltpu.async_copy(sk_hbm.at[srcb.at[j]], ssb[b], sem_g[b])

        def wait_score_gathers(j, b):
            pltpu.make_async_copy(sq_hbm.at[dstb.at[j]], sdb[b], sem_g[b]).wait()
            pltpu.make_async_copy(sk_hbm.at[srcb.at[j]], ssb[b], sem_g[b]).wait()

        def wait_row_gather(j, b):
            pltpu.make_async_copy(xw_hbm.at[rowidxb.at[j]], rowsb[b], sem_g[b]).wait()

        def issue_scatters(j, b):
            pltpu.async_copy(exb[b], ssum_sh.at[dstb.at[j]], sem_s[b], add=True)
            pltpu.async_copy(rowsb[b], y_sh.at[dstb.at[j]], sem_s[b], add=True)

        def drain_scatters(j, b):
            pltpu.make_async_copy(exb[b], ssum_sh.at[dstb.at[j]], sem_s[b]).wait()
            pltpu.make_async_copy(rowsb[b], y_sh.at[dstb.at[j]], sem_s[b]).wait()

        def compute_ex(j, b):
            for j2 in range(_CB // _LANES):
                sl = pl.ds(j2 * _LANES, _LANES)
                iv = lane + j2 * _LANES
                etv = etb[j, sl]
                sqv = plsc.load_gather(sdb[b], [iv, etv])
                skv = plsc.load_gather(ssb[b], [iv, etv])
                al = sqv + skv
                al = jnp.where(al >= 0.0, al, al * 0.2)
                exb[b][sl] = jnp.exp(al)

        def compute_scale(j, b):
            @plsc.parallel_loop(0, _CB, 1, unroll=4)
            def _(i):
                a = plsc.load_gather(exb[b], [jnp.broadcast_to(i, (_LANES,))])
                for h in range(D // _LANES):
                    hsl = pl.ds(h * _LANES, _LANES)
                    rowsb[b][i, hsl] = rowsb[b][i, hsl] * a

        def step(j, b, drain_j, gather_j):
            wait_score_gathers(j, b)
            compute_ex(j, b)
            wait_row_gather(j, b)
            compute_scale(j, b)
            issue_scatters(j, b)
            if drain_j is not None:
                drain_scatters(drain_j, (3 + (b - 1)) % 3)
            if gather_j is not None:
                issue_gathers(gather_j, (3 + (b - 1)) % 3)

        def blk_body(bnum, carry):
            rbase = wid * rows_per_tile + bnum * _NSUB
            pltpu.sync_copy(src_hbm.at[pl.ds(rbase, _NSUB)], srcb)
            pltpu.sync_copy(dst_hbm.at[pl.ds(rbase, _NSUB)], dstb)
            pltpu.sync_copy(et_hbm.at[pl.ds(rbase, _NSUB)], etb)

            issue_gathers(0, 0)
            issue_gathers(1, 1)
            step(0, 0, None, 2)
            step(1, 1, 0, 3)
            step(2, 2, 1, 4)

            def ring_body(k, carry2):
                j0 = 3 * k
                step(j0, 0, j0 - 1, j0 + 2)
                step(j0 + 1, 1, j0, j0 + 3)
                step(j0 + 2, 2, j0 + 1, j0 + 4)
                return carry2
            lax.fori_loop(1, 1 + (_NSUB - 5) // 3, ring_body, 0)

            step(21, 0, 20, 23)
            step(22, 1, 21, 24)
            step(23, 2, 22, None)
            step(24, 0, 23, None)
            drain_scatters(24, 0)
            return carry
        lax.fori_loop(0, nblocks, blk_body, 0)

        plsc.subcore_barrier()

        for t in range(nrows_out // out_r):
            r0 = s * nrows_out + t * out_r
            pltpu.sync_copy(y_sh.at[pl.ds(r0, out_r)], y_out.at[c, pl.ds(r0, out_r)])

        @pl.when(s == 0)
        def _():
            pltpu.sync_copy(ssum_sh, ss_out.at[c])

    return agg


def _prep(x, W, q, k):
    N, D = x.shape
    R = W.shape[0]
    bn = 1000

    def kern(x_ref, w_ref, q_ref, k_ref, xw_ref, sq_ref, sk_ref):
        xb = x_ref[...]
        sq_cols = []
        sk_cols = []
        for r in range(R):
            xwr = jnp.dot(xb, w_ref[r], preferred_element_type=jnp.float32)
            xw_ref[r] = xwr
            sq_cols.append(jnp.dot(xwr, q_ref[...], preferred_element_type=jnp.float32))
            sk_cols.append(jnp.dot(xwr, k_ref[...], preferred_element_type=jnp.float32))
        sq_ref[...] = jnp.concatenate(sq_cols, axis=1)
        sk_ref[...] = jnp.concatenate(sk_cols, axis=1)

    xw, sq, sk = pl.pallas_call(
        kern,
        grid=(N // bn,),
        in_specs=[
            pl.BlockSpec((bn, D), lambda i: (i, 0)),
            pl.BlockSpec((R, D, D), lambda i: (0, 0, 0)),
            pl.BlockSpec((D, 1), lambda i: (0, 0)),
            pl.BlockSpec((D, 1), lambda i: (0, 0)),
        ],
        out_specs=[
            pl.BlockSpec((R, bn, D), lambda i: (0, i, 0)),
            pl.BlockSpec((bn, R), lambda i: (i, 0)),
            pl.BlockSpec((bn, R), lambda i: (i, 0)),
        ],
        out_shape=[
            jax.ShapeDtypeStruct((R, N, D), jnp.float32),
            jax.ShapeDtypeStruct((N, R), jnp.float32),
            jax.ShapeDtypeStruct((N, R), jnp.float32),
        ],
    )(x, W, q, k)
    return xw, sq, sk


def _finish(y_part, ss_part, b):
    _, N, D = y_part.shape
    bn = 1000

    def kern(y_ref, ss_ref, b_ref, o_ref):
        ssum = ss_ref[0, 0] + ss_ref[0, 1]
        denom = ssum + 1e-16
        y = y_ref[0] + y_ref[1]
        o = y / denom[:, None] + b_ref[...][None, :]
        o_ref[...] = jnp.maximum(o, 0.0)

    return pl.pallas_call(
        kern,
        grid=(N // bn,),
        in_specs=[
            pl.BlockSpec((2, bn, D), lambda i: (0, i, 0)),
            pl.BlockSpec((1, 2, bn), lambda i: (i, 0, 0)),
            pl.BlockSpec((D,), lambda i: (0,)),
        ],
        out_specs=pl.BlockSpec((bn, D), lambda i: (i, 0)),
        out_shape=jax.ShapeDtypeStruct((N, D), jnp.float32),
    )(y_part, ss_part.reshape(2, N // bn, bn).transpose(1, 0, 2), b)


def _finish_prep(y_part, ss_part, bb, x, W, q, k):
    _, N, D = y_part.shape
    R = W.shape[0]
    bn = 1000

    def kern(y_ref, ss_ref, b_ref, x_ref, w_ref, q_ref, k_ref,
             o_ref, xw_ref, sq_ref, sk_ref):
        ssum = ss_ref[0, 0] + ss_ref[0, 1]
        denom = ssum + 1e-16
        y = y_ref[0] + y_ref[1]
        o = y / denom[:, None] + b_ref[...][None, :]
        o_ref[...] = jnp.maximum(o, 0.0)
        xb = x_ref[...]
        sq_cols = []
        sk_cols = []
        for r in range(R):
            xwr = jnp.dot(xb, w_ref[r], preferred_element_type=jnp.float32)
            xw_ref[r] = xwr
            sq_cols.append(jnp.dot(xwr, q_ref[...], preferred_element_type=jnp.float32))
            sk_cols.append(jnp.dot(xwr, k_ref[...], preferred_element_type=jnp.float32))
        sq_ref[...] = jnp.concatenate(sq_cols, axis=1)
        sk_ref[...] = jnp.concatenate(sk_cols, axis=1)

    return pl.pallas_call(
        kern,
        grid=(N // bn,),
        in_specs=[
            pl.BlockSpec((2, bn, D), lambda i: (0, i, 0)),
            pl.BlockSpec((1, 2, bn), lambda i: (i, 0, 0)),
            pl.BlockSpec((D,), lambda i: (0,)),
            pl.BlockSpec((bn, D), lambda i: (i, 0)),
            pl.BlockSpec((R, D, D), lambda i: (0, 0, 0)),
            pl.BlockSpec((D, 1), lambda i: (0, 0)),
            pl.BlockSpec((D, 1), lambda i: (0, 0)),
        ],
        out_specs=[
            pl.BlockSpec((bn, D), lambda i: (i, 0)),
            pl.BlockSpec((R, bn, D), lambda i: (0, i, 0)),
            pl.BlockSpec((bn, R), lambda i: (i, 0)),
            pl.BlockSpec((bn, R), lambda i: (i, 0)),
        ],
        out_shape=[
            jax.ShapeDtypeStruct((N, D), jnp.float32),
            jax.ShapeDtypeStruct((R, N, D), jnp.float32),
            jax.ShapeDtypeStruct((N, R), jnp.float32),
            jax.ShapeDtypeStruct((N, R), jnp.float32),
        ],
    )(y_part, ss_part.reshape(2, N // bn, bn).transpose(1, 0, 2), bb,
      x, W, q, k)


def _edges(edge_index, edge_type):
    E = edge_index.shape[1]
    rows = E // _CB
    return (edge_index[0].reshape(rows, _CB), edge_index[1].reshape(rows, _CB),
            edge_type.reshape(rows, _CB))


def _conv(x, edge_index, edge_type, W, q, k, b):
    N, D = x.shape
    E = edge_index.shape[1]
    R = W.shape[0]
    xw, sq, sk = _prep(x, W, q, k)
    rows = E // _CB
    src2 = edge_index[0].reshape(rows, _CB)
    dst2 = edge_index[1].reshape(rows, _CB)
    et2 = edge_type.reshape(rows, _CB)
    y_part, ss_part = _sc_agg(N, E, R, D)(
        src2, dst2, et2, sq, sk, xw.reshape(R * N, D))
    return _finish(y_part, ss_part, b)


def kernel(x_one, edge_index_one, edge_type_one, x_two, edge_index_two,
           edge_type_two, W1, q1, k1, b1, W2, q2, k2, b2):
    h1 = _conv(x_one, edge_index_one, edge_type_one, W1, q1, k1, b1)
    h2 = _conv(x_two, edge_index_two, edge_type_two, W2, q2, k2, b2)
    return (h1, h2)


# speedup vs baseline: 1.1212x; 1.0183x over previous
import functools

import jax
import jax.numpy as jnp
from jax import lax
from jax.experimental import pallas as pl
from jax.experimental.pallas import tpu as pltpu
from jax.experimental.pallas import tpu_sc as plsc

_CB = 80
_NSUB = 25
_NCORES = 2
_NSUB_CORES = 16
_LANES = 16


@functools.lru_cache(maxsize=None)
def _sc_agg(N, E, R, D):
    nw = _NCORES * _NSUB_CORES
    rows_total = E // _CB
    rows_per_tile = rows_total // nw
    nblocks = rows_per_tile // _NSUB
    nrows_out = N // _NSUB_CORES
    zr = 25
    out_r = 125

    mesh = plsc.VectorSubcoreMesh(core_axis_name="c", subcore_axis_name="s")

    @functools.partial(
        pl.kernel,
        out_type=[
            jax.ShapeDtypeStruct((_NCORES, N, D), jnp.float32),
            jax.ShapeDtypeStruct((_NCORES, N), jnp.float32),
        ],
        mesh=mesh,
        compiler_params=pltpu.CompilerParams(
            use_tc_tiling_on_sc=False, needs_layout_passes=False),
        scratch_types=[
            pltpu.VMEM((_NSUB, _CB), jnp.int32),
            pltpu.VMEM((_NSUB, _CB), jnp.int32),
            pltpu.VMEM((_NSUB, _CB), jnp.int32),
            pltpu.VMEM((_NSUB, _CB), jnp.int32),
            pltpu.VMEM((_CB,), jnp.float32),
            pltpu.VMEM((_CB,), jnp.float32),
            pltpu.VMEM((_CB,), jnp.float32),
            pltpu.VMEM((_CB, D), jnp.float32),
            pltpu.VMEM((_CB, D), jnp.float32),
            pltpu.VMEM((_CB, D), jnp.float32),
            pltpu.VMEM((_CB, R), jnp.float32),
            pltpu.VMEM((_CB, R), jnp.float32),
            pltpu.VMEM((_CB, R), jnp.float32),
            pltpu.VMEM((_CB, R), jnp.float32),
            pltpu.VMEM((_CB, R), jnp.float32),
            pltpu.VMEM((_CB, R), jnp.float32),
            pltpu.VMEM((zr, D), jnp.float32),
            pltpu.VMEM((2000,), jnp.float32),
            pltpu.VMEM_SHARED((N, D), jnp.float32),
            pltpu.VMEM_SHARED((N,), jnp.float32),
            pltpu.SemaphoreType.DMA,
            pltpu.SemaphoreType.DMA,
            pltpu.SemaphoreType.DMA,
            pltpu.SemaphoreType.DMA,
            pltpu.SemaphoreType.DMA,
            pltpu.SemaphoreType.DMA,
            pltpu.SemaphoreType.DMA,
            pltpu.SemaphoreType.DMA,
            pltpu.SemaphoreType.DMA,
        ],
    )
    def agg(src_hbm, dst_hbm, et_hbm, sq_hbm, sk_hbm, xw_hbm, y_out, ss_out,
            srcb, dstb, etb, rowidxb, exb0, exb1, exb2,
            rowsb0, rowsb1, rowsb2, sdb0, sdb1, sdb2, ssb0, ssb1, ssb2,
            zbuf2, zbuf1, y_sh, ssum_sh,
            sem_g0, sem_g1, sem_g2, sem_s0, sem_s1, sem_s2,
            sem_c0, sem_c1, sem_c2):
        exb = [exb0, exb1, exb2]
        rowsb = [rowsb0, rowsb1, rowsb2]
        sdb = [sdb0, sdb1, sdb2]
        ssb = [ssb0, ssb1, ssb2]
        sem_g = [sem_g0, sem_g1, sem_g2]
        sem_s = [sem_s0, sem_s1, sem_s2]
        sem_c = [sem_c0, sem_c1, sem_c2]
        c = lax.axis_index("c")
        s = lax.axis_index("s")
        wid = c * _NSUB_CORES + s
        lane = lax.iota(jnp.int32, _LANES)

        zv = jnp.zeros((_LANES,), jnp.float32)

        def _z2(i, carry):
            zbuf2[lax.div(i, 8), pl.ds(lax.rem(i, 8) * _LANES, _LANES)] = zv
            return carry
        lax.fori_loop(0, zr * (D // _LANES), _z2, 0)

        def _z1(i, carry):
            zbuf1[pl.ds(i * _LANES, _LANES)] = zv
            return carry
        lax.fori_loop(0, 2000 // _LANES, _z1, 0)

        for t in range(nrows_out // zr):
            pltpu.sync_copy(zbuf2, y_sh.at[pl.ds(s * nrows_out + t * zr, zr)])

        @pl.when(s == 0)
        def _():
            for t in range(N // 2000):
                pltpu.sync_copy(zbuf1, ssum_sh.at[pl.ds(t * 2000, 2000)])

        plsc.subcore_barrier()

        def issue_gathers(j, b):
            for j2 in range(_CB // _LANES):
                sl = pl.ds(j2 * _LANES, _LANES)
                rowidxb[j, sl] = etb[j, sl] * N + srcb[j, sl]
            pltpu.async_copy(xw_hbm.at[rowidxb.at[j]], rowsb[b], sem_g[b])
            pltpu.async_copy(sq_hbm.at[dstb.at[j]], sdb[b], sem_c[b])
            pltpu.async_copy(sk_hbm.at[srcb.at[j]], ssb[b], sem_c[b])

        def wait_score_gathers(j, b):
            pltpu.make_async_copy(sq_hbm.at[dstb.at[j]], sdb[b], sem_c[b]).wait()
            pltpu.make_async_copy(sk_hbm.at[srcb.at[j]], ssb[b], sem_c[b]).wait()

        def wait_row_gather(j, b):
            pltpu.make_async_copy(xw_hbm.at[rowidxb.at[j]], rowsb[b], sem_g[b]).wait()

        def issue_scatters(j, b):
            pltpu.async_copy(exb[b], ssum_sh.at[dstb.at[j]], sem_s[b], add=True)
            pltpu.async_copy(rowsb[b], y_sh.at[dstb.at[j]], sem_s[b], add=True)

        def drain_scatters(j, b):
            pltpu.make_async_copy(exb[b], ssum_sh.at[dstb.at[j]], sem_s[b]).wait()
            pltpu.make_async_copy(rowsb[b], y_sh.at[dstb.at[j]], sem_s[b]).wait()

        def compute_ex(j, b):
            for j2 in range(_CB // _LANES):
                sl = pl.ds(j2 * _LANES, _LANES)
                iv = lane + j2 * _LANES
                etv = etb[j, sl]
                sqv = plsc.load_gather(sdb[b], [iv, etv])
                skv = plsc.load_gather(ssb[b], [iv, etv])
                al = sqv + skv
                al = jnp.where(al >= 0.0, al, al * 0.2)
                exb[b][sl] = jnp.exp(al)

        def compute_scale(j, b):
            @plsc.parallel_loop(0, _CB, 1, unroll=4)
            def _(i):
                a = plsc.load_gather(exb[b], [jnp.broadcast_to(i, (_LANES,))])
                for h in range(D // _LANES):
                    hsl = pl.ds(h * _LANES, _LANES)
                    rowsb[b][i, hsl] = rowsb[b][i, hsl] * a

        def step(j, b, drain_j, gather_j):
            wait_score_gathers(j, b)
            compute_ex(j, b)
            wait_row_gather(j, b)
            compute_scale(j, b)
            issue_scatters(j, b)
            if drain_j is not None:
                drain_scatters(drain_j, (3 + (b - 1)) % 3)
            if gather_j is not None:
                issue_gathers(gather_j, (3 + (b - 1)) % 3)

        def blk_body(bnum, carry):
            rbase = wid * rows_per_tile + bnum * _NSUB
            pltpu.sync_copy(src_hbm.at[pl.ds(rbase, _NSUB)], srcb)
            pltpu.sync_copy(dst_hbm.at[pl.ds(rbase, _NSUB)], dstb)
            pltpu.sync_copy(et_hbm.at[pl.ds(rbase, _NSUB)], etb)

            issue_gathers(0, 0)
            issue_gathers(1, 1)
            step(0, 0, None, 2)
            step(1, 1, 0, 3)
            step(2, 2, 1, 4)

            def ring_body(k, carry2):
                j0 = 3 * k
                step(j0, 0, j0 - 1, j0 + 2)
                step(j0 + 1, 1, j0, j0 + 3)
                step(j0 + 2, 2, j0 + 1, j0 + 4)
                return carry2
            lax.fori_loop(1, 1 + (_NSUB - 5) // 3, ring_body, 0)

            step(21, 0, 20, 23)
            step(22, 1, 21, 24)
            step(23, 2, 22, None)
            step(24, 0, 23, None)
            drain_scatters(24, 0)
            return carry
        lax.fori_loop(0, nblocks, blk_body, 0)

        plsc.subcore_barrier()

        for t in range(nrows_out // out_r):
            r0 = s * nrows_out + t * out_r
            pltpu.sync_copy(y_sh.at[pl.ds(r0, out_r)], y_out.at[c, pl.ds(r0, out_r)])

        @pl.when(s == 0)
        def _():
            pltpu.sync_copy(ssum_sh, ss_out.at[c])

    return agg


def _prep(x, W, q, k):
    N, D = x.shape
    R = W.shape[0]
    bn = 1000

    def kern(x_ref, w_ref, q_ref, k_ref, xw_ref, sq_ref, sk_ref):
        xb = x_ref[...]
        sq_cols = []
        sk_cols = []
        for r in range(R):
            xwr = jnp.dot(xb, w_ref[r], preferred_element_type=jnp.float32)
            xw_ref[r] = xwr
            sq_cols.append(jnp.dot(xwr, q_ref[...], preferred_element_type=jnp.float32))
            sk_cols.append(jnp.dot(xwr, k_ref[...], preferred_element_type=jnp.float32))
        sq_ref[...] = jnp.concatenate(sq_cols, axis=1)
        sk_ref[...] = jnp.concatenate(sk_cols, axis=1)

    xw, sq, sk = pl.pallas_call(
        kern,
        grid=(N // bn,),
        in_specs=[
            pl.BlockSpec((bn, D), lambda i: (i, 0)),
            pl.BlockSpec((R, D, D), lambda i: (0, 0, 0)),
            pl.BlockSpec((D, 1), lambda i: (0, 0)),
            pl.BlockSpec((D, 1), lambda i: (0, 0)),
        ],
        out_specs=[
            pl.BlockSpec((R, bn, D), lambda i: (0, i, 0)),
            pl.BlockSpec((bn, R), lambda i: (i, 0)),
            pl.BlockSpec((bn, R), lambda i: (i, 0)),
        ],
        out_shape=[
            jax.ShapeDtypeStruct((R, N, D), jnp.float32),
            jax.ShapeDtypeStruct((N, R), jnp.float32),
            jax.ShapeDtypeStruct((N, R), jnp.float32),
        ],
    )(x, W, q, k)
    return xw, sq, sk


def _finish(y_part, ss_part, b):
    _, N, D = y_part.shape
    bn = 1000

    def kern(y_ref, ss_ref, b_ref, o_ref):
        ssum = ss_ref[0, 0] + ss_ref[0, 1]
        denom = ssum + 1e-16
        y = y_ref[0] + y_ref[1]
        o = y / denom[:, None] + b_ref[...][None, :]
        o_ref[...] = jnp.maximum(o, 0.0)

    return pl.pallas_call(
        kern,
        grid=(N // bn,),
        in_specs=[
            pl.BlockSpec((2, bn, D), lambda i: (0, i, 0)),
            pl.BlockSpec((1, 2, bn), lambda i: (i, 0, 0)),
            pl.BlockSpec((D,), lambda i: (0,)),
        ],
        out_specs=pl.BlockSpec((bn, D), lambda i: (i, 0)),
        out_shape=jax.ShapeDtypeStruct((N, D), jnp.float32),
    )(y_part, ss_part.reshape(2, N // bn, bn).transpose(1, 0, 2), b)


def _finish_prep(y_part, ss_part, bb, x, W, q, k):
    _, N, D = y_part.shape
    R = W.shape[0]
    bn = 1000

    def kern(y_ref, ss_ref, b_ref, x_ref, w_ref, q_ref, k_ref,
             o_ref, xw_ref, sq_ref, sk_ref):
        ssum = ss_ref[0, 0] + ss_ref[0, 1]
        denom = ssum + 1e-16
        y = y_ref[0] + y_ref[1]
        o = y / denom[:, None] + b_ref[...][None, :]
        o_ref[...] = jnp.maximum(o, 0.0)
        xb = x_ref[...]
        sq_cols = []
        sk_cols = []
        for r in range(R):
            xwr = jnp.dot(xb, w_ref[r], preferred_element_type=jnp.float32)
            xw_ref[r] = xwr
            sq_cols.append(jnp.dot(xwr, q_ref[...], preferred_element_type=jnp.float32))
            sk_cols.append(jnp.dot(xwr, k_ref[...], preferred_element_type=jnp.float32))
        sq_ref[...] = jnp.concatenate(sq_cols, axis=1)
        sk_ref[...] = jnp.concatenate(sk_cols, axis=1)

    return pl.pallas_call(
        kern,
        grid=(N // bn,),
        in_specs=[
            pl.BlockSpec((2, bn, D), lambda i: (0, i, 0)),
            pl.BlockSpec((1, 2, bn), lambda i: (i, 0, 0)),
            pl.BlockSpec((D,), lambda i: (0,)),
            pl.BlockSpec((bn, D), lambda i: (i, 0)),
            pl.BlockSpec((R, D, D), lambda i: (0, 0, 0)),
            pl.BlockSpec((D, 1), lambda i: (0, 0)),
            pl.BlockSpec((D, 1), lambda i: (0, 0)),
        ],
        out_specs=[
            pl.BlockSpec((bn, D), lambda i: (i, 0)),
            pl.BlockSpec((R, bn, D), lambda i: (0, i, 0)),
            pl.BlockSpec((bn, R), lambda i: (i, 0)),
            pl.BlockSpec((bn, R), lambda i: (i, 0)),
        ],
        out_shape=[
            jax.ShapeDtypeStruct((N, D), jnp.float32),
            jax.ShapeDtypeStruct((R, N, D), jnp.float32),
            jax.ShapeDtypeStruct((N, R), jnp.float32),
            jax.ShapeDtypeStruct((N, R), jnp.float32),
        ],
    )(y_part, ss_part.reshape(2, N // bn, bn).transpose(1, 0, 2), bb,
      x, W, q, k)


def _edges(edge_index, edge_type):
    E = edge_index.shape[1]
    rows = E // _CB
    return (edge_index[0].reshape(rows, _CB), edge_index[1].reshape(rows, _CB),
            edge_type.reshape(rows, _CB))


def _conv(x, edge_index, edge_type, W, q, k, b):
    N, D = x.shape
    E = edge_index.shape[1]
    R = W.shape[0]
    xw, sq, sk = _prep(x, W, q, k)
    rows = E // _CB
    src2 = edge_index[0].reshape(rows, _CB)
    dst2 = edge_index[1].reshape(rows, _CB)
    et2 = edge_type.reshape(rows, _CB)
    y_part, ss_part = _sc_agg(N, E, R, D)(
        src2, dst2, et2, sq, sk, xw.reshape(R * N, D))
    return _finish(y_part, ss_part, b)


def kernel(x_one, edge_index_one, edge_type_one, x_two, edge_index_two,
           edge_type_two, W1, q1, k1, b1, W2, q2, k2, b2):
    h1 = _conv(x_one, edge_index_one, edge_type_one, W1, q1, k1, b1)
    h2 = _conv(x_two, edge_index_two, edge_type_two, W2, q2, k2, b2)
    return (h1, h2)
